# baseline (speedup 1.0000x reference)
import jax
import jax.numpy as jnp
from jax import lax
from jax.experimental import pallas as pl
from jax.experimental.pallas import tpu as pltpu
from jax.experimental.pallas import tpu_sc as plsc

_N = 10000
_E = 160000
_D = 128
_DE = 16
_G = 64
_C = 10

_ROW = 512
_RSL = 2
_NP = 10240
_NB = 16
_BN = _NP // _NB

_CHUNK = 64
_NCHUNKS = _E // _CHUNK
_NWORKERS = 32
_CHB = _NCHUNKS // _NWORKERS
_CHR = _NCHUNKS % _NWORKERS
_MAXCH = _CHB + 1
_EA_RPC = _CHUNK * _DE // 128
_EPAD = _E + _CHUNK


def _stage_a_body(x_ref, a_ref, root_ref, bias_ref, t_ref, r_ref):
    x = x_ref[...]
    t_ref[...] = jnp.dot(
        x, a_ref[...], preferred_element_type=jnp.float32
    ).astype(jnp.bfloat16)
    r_ref[...] = (
        jnp.dot(x, root_ref[...], preferred_element_type=jnp.float32)
        + bias_ref[...]
    )


def _stage_a(x, a1, root1p, bias1p):
    return pl.pallas_call(
        _stage_a_body,
        grid=(_NB,),
        in_specs=[
            pl.BlockSpec((_BN, _D), lambda i: (i, 0)),
            pl.BlockSpec((_D, _ROW), lambda i: (0, 0)),
            pl.BlockSpec((_D, 16), lambda i: (0, 0)),
            pl.BlockSpec((1, 16), lambda i: (0, 0)),
        ],
        out_specs=[
            pl.BlockSpec((_BN, _ROW), lambda i: (i, 0)),
            pl.BlockSpec((_BN, 16), lambda i: (i, 0)),
        ],
        out_shape=[
            jax.ShapeDtypeStruct((_NP, _ROW), jnp.bfloat16),
            jax.ShapeDtypeStruct((_NP, 16), jnp.float32),
        ],
    )(x, a1, root1p, bias1p)


def _stage_b_body(a0_ref, a1_ref, r_ref, an_ref, rootn_ref, biasn_ref,
                  t_ref, rn_ref):
    h = jax.nn.relu(a0_ref[...] + a1_ref[...] + r_ref[...])
    t_ref[...] = jnp.dot(
        h, an_ref[...], preferred_element_type=jnp.float32
    ).astype(jnp.bfloat16)
    rn_ref[...] = (
        jnp.dot(h, rootn_ref[...], preferred_element_type=jnp.float32)
        + biasn_ref[...]
    )


def _stage_b(agg0, agg1, r, a_next, root_next, bias_next):
    return pl.pallas_call(
        _stage_b_body,
        grid=(_NB,),
        in_specs=[
            pl.BlockSpec((_BN, 16), lambda i: (i, 0)),
            pl.BlockSpec((_BN, 16), lambda i: (i, 0)),
            pl.BlockSpec((_BN, 16), lambda i: (i, 0)),
            pl.BlockSpec((16, _ROW), lambda i: (0, 0)),
            pl.BlockSpec((16, 16), lambda i: (0, 0)),
            pl.BlockSpec((1, 16), lambda i: (0, 0)),
        ],
        out_specs=[
            pl.BlockSpec((_BN, _ROW), lambda i: (i, 0)),
            pl.BlockSpec((_BN, 16), lambda i: (i, 0)),
        ],
        out_shape=[
            jax.ShapeDtypeStruct((_NP, _ROW), jnp.bfloat16),
            jax.ShapeDtypeStruct((_NP, 16), jnp.float32),
        ],
    )(agg0, agg1, r, a_next, root_next, bias_next)


def _stage_c_body(a0_ref, a1_ref, r_ref, batch_ref, wl_ref, bl_ref,
                  out_ref, pooled_ref):
    i = pl.program_id(0)

    @pl.when(i == 0)
    def _init():
        pooled_ref[...] = jnp.zeros_like(pooled_ref)

    h = jax.nn.relu(a0_ref[...] + a1_ref[...] + r_ref[...])
    b = batch_ref[0, 0, :]
    seg = lax.broadcasted_iota(jnp.int32, (_G, _BN), 0)
    mask = (seg == b[None, :]).astype(jnp.float32)
    pooled_ref[...] += jnp.dot(mask, h, preferred_element_type=jnp.float32)

    @pl.when(i == _NB - 1)
    def _fin():
        out_ref[...] = (
            jnp.dot(pooled_ref[...], wl_ref[...],
                    preferred_element_type=jnp.float32)
            + bl_ref[...]
        )


def _stage_c(agg0, agg1, r, batch3d, w_lin, b_lin2d):
    return pl.pallas_call(
        _stage_c_body,
        grid=(_NB,),
        in_specs=[
            pl.BlockSpec((_BN, 16), lambda i: (i, 0)),
            pl.BlockSpec((_BN, 16), lambda i: (i, 0)),
            pl.BlockSpec((_BN, 16), lambda i: (i, 0)),
            pl.BlockSpec((1, 1, _BN), lambda i: (i, 0, 0)),
            pl.BlockSpec((16, _C), lambda i: (0, 0)),
            pl.BlockSpec((1, _C), lambda i: (0, 0)),
        ],
        out_specs=pl.BlockSpec((_G, _C), lambda i: (0, 0)),
        out_shape=jax.ShapeDtypeStruct((_G, _C), jnp.float32),
        scratch_shapes=[pltpu.VMEM((_G, 16), jnp.float32)],
    )(agg0, agg1, r, batch3d, w_lin, b_lin2d)


_AROWS = _NP // 8
_AR_TILE = _AROWS // 16
_GROUPS = _CHUNK // 16


def _edge_body(table_hbm, src_hbm, dst_hbm, ea_hbm, out_hbm,
               src_all, dst_all, nbuf_v,
               ea_b, rows_b, msg_b, didx_b, semg, sems,
               agg_sh):
    c = lax.axis_index("c")
    s = lax.axis_index("s")
    wid = c * 16 + s
    z16 = jnp.zeros((16,), jnp.float32)

    nch = _CHB + jnp.where(wid < _CHR, 1, 0)
    cstart = _CHB * wid + jnp.minimum(wid, _CHR)
    ebase = pl.multiple_of(cstart * _CHUNK, _CHUNK)

    def _zrow(j, carry):
        for t in range(8):
            nbuf_v[j, 16 * t: 16 * t + 16] = z16
        return carry
    lax.fori_loop(0, _AR_TILE, _zrow, 0)
    pltpu.sync_copy(nbuf_v, agg_sh.at[pl.ds(s * _AR_TILE, _AR_TILE)])
    pltpu.sync_copy(src_hbm.at[pl.ds(ebase, _MAXCH * _CHUNK)], src_all)
    pltpu.sync_copy(dst_hbm.at[pl.ds(ebase, _MAXCH * _CHUNK)], dst_all)
    plsc.subcore_barrier()

    def _start_in(i, b):
        ea_off = pl.multiple_of((cstart + i) * _EA_RPC, _EA_RPC)
        pltpu.async_copy(ea_hbm.at[pl.ds(ea_off, _EA_RPC), :], ea_b[b],
                         semg[b])
        pltpu.async_copy(
            table_hbm.at[src_all.at[pl.ds(i * _CHUNK, _CHUNK)]],
            rows_b[b], semg[b])

    def _wait_in(b):
        pltpu.make_async_copy(ea_hbm.at[pl.ds(0, _EA_RPC), :], ea_b[b],
                              semg[b]).wait()
        pltpu.make_async_copy(table_hbm.at[pl.ds(0, _CHUNK)], rows_b[b],
                              semg[b]).wait()

    def _pair_slots(rows_v, j, t):
        q = 16 * t
        raw = rows_v[j, q // 128, q % 128: q % 128 + 16]
        va = lax.bitcast_convert_type(lax.shift_left(raw, 16), jnp.float32)
        vb = lax.bitcast_convert_type(raw & jnp.int32(-65536), jnp.float32)
        return va, vb

    def _wait_scat(b):
        pltpu.make_async_copy(msg_b[b], agg_sh.at[didx_b[b]],
                              sems[b]).wait()

    def _compute(i, b):
        rows_v = rows_b[b]
        ea_v = ea_b[b]
        msg_v = msg_b[b]
        didx_v = didx_b[b]

        def _group(g, gcarry):
            dgrp = dst_all[pl.ds(i * _CHUNK + 16 * g, 16)]
            didx_v[pl.ds(16 * g, 16)] = lax.shift_right_logical(dgrp, 3)
            slot = (dgrp & 7) * 16
            for e in range(16):
                j = 16 * g + e
                arow = ea_v[2 * g + e // 8, (e % 8) * 16: (e % 8) * 16 + 16]
                va, vb = _pair_slots(rows_v, j, 0)
                m = va + arow[0] * vb
                for t in range(1, 8):
                    va, vb = _pair_slots(rows_v, j, t)
                    m = m + arow[2 * t - 1] * va + arow[2 * t] * vb
                va, _ = _pair_slots(rows_v, j, 8)
                m = m + arow[15] * va
                for t in range(8):
                    msg_v[j, 16 * t: 16 * t + 16] = z16
                msg_v[j, pl.ds(slot[e], 16)] = m
            return gcarry
        lax.fori_loop(0, _GROUPS, _group, 0)
        pltpu.async_copy(msg_v, agg_sh.at[didx_v], sems[b], add=True)

    _start_in(0, 0)

    def _pair(i2, carry):
        a = 2 * i2
        b = a + 1

        @pl.when(b < nch)
        def _():
            _start_in(b, 1)
        _wait_in(0)

        @pl.when(i2 >= 1)
        def _():
            _wait_scat(0)
        _compute(a, 0)

        @pl.when(a + 2 < nch)
        def _():
            _start_in(a + 2, 0)

        @pl.when(b < nch)
        def _():
            _wait_in(1)

            @pl.when(i2 >= 1)
            def _():
                _wait_scat(1)
            _compute(b, 1)
        return carry

    lax.fori_loop(0, (nch + 1) // 2, _pair, 0)
    _wait_scat(0)
    _wait_scat(1)
    plsc.subcore_barrier()

    pltpu.sync_copy(agg_sh.at[pl.ds(s * _AR_TILE, _AR_TILE)], nbuf_v)
    pltpu.sync_copy(nbuf_v, out_hbm.at[c, pl.ds(s * _AR_TILE, _AR_TILE)])


def _edge_pass(table, src, dst, ea2):
    mesh = plsc.VectorSubcoreMesh(core_axis_name="c", subcore_axis_name="s")
    f = pl.kernel(
        _edge_body,
        out_type=jax.ShapeDtypeStruct((2, _AROWS, 128), jnp.float32),
        mesh=mesh,
        scratch_types=[
            pltpu.VMEM((_MAXCH * _CHUNK,), jnp.int32),
            pltpu.VMEM((_MAXCH * _CHUNK,), jnp.int32),
            pltpu.VMEM((_AR_TILE, 128), jnp.float32),
            [pltpu.VMEM((_EA_RPC, 128), jnp.float32) for _ in range(2)],
            [pltpu.VMEM((_CHUNK, _RSL, 128), jnp.int32) for _ in range(2)],
            [pltpu.VMEM((_CHUNK, 128), jnp.float32) for _ in range(2)],
            [pltpu.VMEM((_CHUNK,), jnp.int32) for _ in range(2)],
            [pltpu.SemaphoreType.DMA for _ in range(2)],
            [pltpu.SemaphoreType.DMA for _ in range(2)],
            pltpu.VMEM_SHARED((_AROWS, 128), jnp.float32),
        ],
    )
    out = f(table, src, dst, ea2)
    return out.reshape(2, _NP, 16)


def _build_a(w_e, b_e, cin, cout):
    w = w_e.reshape(_DE, cin, cout).transpose(1, 0, 2)
    w = jnp.pad(w, ((0, 0), (0, 0), (0, 16 - cout)))
    b = jnp.pad(b_e.reshape(cin, cout), ((0, 0), (0, 16 - cout)))
    slots = jnp.concatenate(
        [b[:, None, :], w, jnp.zeros((cin, 1, 16), jnp.float32)], axis=1
    )
    pairs = slots.reshape(cin, 9, 2, 16).transpose(0, 1, 3, 2)
    a = pairs.reshape(cin, 9 * 32)
    return jnp.pad(a, ((0, 0), (0, _ROW - a.shape[1])))


def _pad_rows(m, rows):
    return jnp.pad(m, ((0, rows - m.shape[0]), (0, 0)))


def kernel(x, edge_index, edge_attr, batch,
           W_e1, b_e1, root1, bias1,
           W_e2, b_e2, root2, bias2,
           W_e3, b_e3, root3, bias3,
           W_lin, b_lin):
    src = edge_index[0]
    dst = edge_index[1]

    a1 = _build_a(W_e1, b_e1, _D, 8)
    root1p = jnp.pad(root1, ((0, 0), (0, 8)))
    bias1p = jnp.pad(bias1, (0, 8)).reshape(1, 16)

    a2 = _pad_rows(_build_a(W_e2, b_e2, 8, 16), 16)
    root2p = _pad_rows(root2, 16)
    bias2p = bias2.reshape(1, 16)

    a3 = _build_a(W_e3, b_e3, 16, 16)
    bias3p = bias3.reshape(1, 16)

    x_pad = jnp.pad(x, ((0, _NP - _N), (0, 0)))
    batch_pad = jnp.concatenate(
        [batch, jnp.full((_NP - _N,), _G, jnp.int32)])
    batch3d = batch_pad.reshape(_NB, 1, _BN)
    npad = _EPAD - _E
    src = jnp.concatenate([src, jnp.zeros((npad,), jnp.int32)])
    dst = jnp.concatenate([dst, jnp.zeros((npad,), jnp.int32)])
    ea2 = jnp.concatenate(
        [edge_attr, jnp.zeros((npad, _DE), jnp.float32)]
    ).reshape(_EPAD * _DE // 128, 128)
    b_lin2d = b_lin.reshape(1, _C)

    def _as_i32(t):
        return lax.bitcast_convert_type(
            t.reshape(_NP, _ROW // 2, 2), jnp.int32).reshape(_NP, _RSL, 128)

    t1, r1 = _stage_a(x_pad, a1, root1p, bias1p)
    agg1 = _edge_pass(_as_i32(t1), src, dst, ea2)
    t2, r2 = _stage_b(agg1[0], agg1[1], r1, a2, root2p, bias2p)
    agg2 = _edge_pass(_as_i32(t2), src, dst, ea2)
    t3, r3 = _stage_b(agg2[0], agg2[1], r2, a3, root3, bias3p)
    agg3 = _edge_pass(_as_i32(t3), src, dst, ea2)
    out = _stage_c(agg3[0], agg3[1], r3, batch3d, W_lin, b_lin2d)
    return out


# speedup vs baseline: 1.5102x; 1.5102x over previous
import jax
import jax.numpy as jnp
from jax import lax
from jax.experimental import pallas as pl
from jax.experimental.pallas import tpu as pltpu
from jax.experimental.pallas import tpu_sc as plsc

_N = 10000
_E = 160000
_D = 128
_DE = 16
_G = 64
_C = 10

_ROW = 512
_RSL = 2
_NP = 10240
_NB = 16
_BN = _NP // _NB

_CHUNK = 64
_NCHUNKS = _E // _CHUNK
_NWORKERS = 32
_CHB = _NCHUNKS // _NWORKERS
_CHR = _NCHUNKS % _NWORKERS
_MAXCH = _CHB + 1
_EA_RPC = _CHUNK * _DE // 128
_EPAD = _E + _CHUNK


def _pack_table(t):
    half = _ROW // 2
    lo = lax.bitcast_convert_type(
        t[:, :half].astype(jnp.bfloat16), jnp.uint16).astype(jnp.uint32)
    hi = lax.bitcast_convert_type(
        t[:, half:].astype(jnp.bfloat16), jnp.uint16).astype(jnp.uint32)
    return lax.bitcast_convert_type(lo | (hi << 16), jnp.int32)

def _stage_a_body(x_ref, a_ref, root_ref, bias_ref, t_ref, r_ref):
    x = x_ref[...]
    t_ref[...] = _pack_table(
        jnp.dot(x, a_ref[...], preferred_element_type=jnp.float32))
    r_ref[...] = (
        jnp.dot(x, root_ref[...], preferred_element_type=jnp.float32)
        + bias_ref[...]
    )


def _stage_a(x, a1, root1p, bias1p):
    return pl.pallas_call(
        _stage_a_body,
        grid=(_NB,),
        in_specs=[
            pl.BlockSpec((_BN, _D), lambda i: (i, 0)),
            pl.BlockSpec((_D, _ROW), lambda i: (0, 0)),
            pl.BlockSpec((_D, 16), lambda i: (0, 0)),
            pl.BlockSpec((1, 16), lambda i: (0, 0)),
        ],
        out_specs=[
            pl.BlockSpec((_BN, _ROW // 2), lambda i: (i, 0)),
            pl.BlockSpec((_BN, 16), lambda i: (i, 0)),
        ],
        out_shape=[
            jax.ShapeDtypeStruct((_NP, _ROW // 2), jnp.int32),
            jax.ShapeDtypeStruct((_NP, 16), jnp.float32),
        ],
    )(x, a1, root1p, bias1p)


def _stage_b_body(a0_ref, a1_ref, r_ref, an_ref, rootn_ref, biasn_ref,
                  t_ref, rn_ref):
    h = jax.nn.relu(a0_ref[...] + a1_ref[...] + r_ref[...])
    t_ref[...] = _pack_table(
        jnp.dot(h, an_ref[...], preferred_element_type=jnp.float32))
    rn_ref[...] = (
        jnp.dot(h, rootn_ref[...], preferred_element_type=jnp.float32)
        + biasn_ref[...]
    )


def _stage_b(agg0, agg1, r, a_next, root_next, bias_next):
    return pl.pallas_call(
        _stage_b_body,
        grid=(_NB,),
        in_specs=[
            pl.BlockSpec((_BN, 16), lambda i: (i, 0)),
            pl.BlockSpec((_BN, 16), lambda i: (i, 0)),
            pl.BlockSpec((_BN, 16), lambda i: (i, 0)),
            pl.BlockSpec((16, _ROW), lambda i: (0, 0)),
            pl.BlockSpec((16, 16), lambda i: (0, 0)),
            pl.BlockSpec((1, 16), lambda i: (0, 0)),
        ],
        out_specs=[
            pl.BlockSpec((_BN, _ROW // 2), lambda i: (i, 0)),
            pl.BlockSpec((_BN, 16), lambda i: (i, 0)),
        ],
        out_shape=[
            jax.ShapeDtypeStruct((_NP, _ROW // 2), jnp.int32),
            jax.ShapeDtypeStruct((_NP, 16), jnp.float32),
        ],
    )(agg0, agg1, r, a_next, root_next, bias_next)


def _stage_c_body(a0_ref, a1_ref, r_ref, batch_ref, wl_ref, bl_ref,
                  out_ref, pooled_ref):
    i = pl.program_id(0)

    @pl.when(i == 0)
    def _init():
        pooled_ref[...] = jnp.zeros_like(pooled_ref)

    h = jax.nn.relu(a0_ref[...] + a1_ref[...] + r_ref[...])
    b = batch_ref[0, 0, :]
    seg = lax.broadcasted_iota(jnp.int32, (_G, _BN), 0)
    mask = (seg == b[None, :]).astype(jnp.float32)
    pooled_ref[...] += jnp.dot(mask, h, preferred_element_type=jnp.float32)

    @pl.when(i == _NB - 1)
    def _fin():
        out_ref[...] = (
            jnp.dot(pooled_ref[...], wl_ref[...],
                    preferred_element_type=jnp.float32)
            + bl_ref[...]
        )


def _stage_c(agg0, agg1, r, batch3d, w_lin, b_lin2d):
    return pl.pallas_call(
        _stage_c_body,
        grid=(_NB,),
        in_specs=[
            pl.BlockSpec((_BN, 16), lambda i: (i, 0)),
            pl.BlockSpec((_BN, 16), lambda i: (i, 0)),
            pl.BlockSpec((_BN, 16), lambda i: (i, 0)),
            pl.BlockSpec((1, 1, _BN), lambda i: (i, 0, 0)),
            pl.BlockSpec((16, _C), lambda i: (0, 0)),
            pl.BlockSpec((1, _C), lambda i: (0, 0)),
        ],
        out_specs=pl.BlockSpec((_G, _C), lambda i: (0, 0)),
        out_shape=jax.ShapeDtypeStruct((_G, _C), jnp.float32),
        scratch_shapes=[pltpu.VMEM((_G, 16), jnp.float32)],
    )(agg0, agg1, r, batch3d, w_lin, b_lin2d)


_AROWS = _NP // 8
_AR_TILE = _AROWS // 16
_GROUPS = _CHUNK // 16


def _edge_body(table_hbm, src_hbm, dst_hbm, ea_hbm, out_hbm,
               src_all, dst_all, nbuf_v,
               ea_b, rows_b, msg_b, didx_b, semg, sems,
               agg_sh):
    c = lax.axis_index("c")
    s = lax.axis_index("s")
    wid = c * 16 + s
    z16 = jnp.zeros((16,), jnp.float32)

    nch = _CHB + jnp.where(wid < _CHR, 1, 0)
    cstart = _CHB * wid + jnp.minimum(wid, _CHR)
    ebase = pl.multiple_of(cstart * _CHUNK, _CHUNK)

    def _zrow(j, carry):
        for t in range(8):
            nbuf_v[j, 16 * t: 16 * t + 16] = z16
        return carry
    lax.fori_loop(0, _AR_TILE, _zrow, 0)
    pltpu.sync_copy(nbuf_v, agg_sh.at[pl.ds(s * _AR_TILE, _AR_TILE)])
    pltpu.sync_copy(src_hbm.at[pl.ds(ebase, _MAXCH * _CHUNK)], src_all)
    pltpu.sync_copy(dst_hbm.at[pl.ds(ebase, _MAXCH * _CHUNK)], dst_all)
    plsc.subcore_barrier()

    def _start_in(i, b):
        ea_off = pl.multiple_of((cstart + i) * _EA_RPC, _EA_RPC)
        pltpu.async_copy(ea_hbm.at[pl.ds(ea_off, _EA_RPC), :], ea_b[b],
                         semg[b])
        pltpu.async_copy(
            table_hbm.at[src_all.at[pl.ds(i * _CHUNK, _CHUNK)]],
            rows_b[b], semg[b])

    def _wait_in(b):
        pltpu.make_async_copy(ea_hbm.at[pl.ds(0, _EA_RPC), :], ea_b[b],
                              semg[b]).wait()
        pltpu.make_async_copy(table_hbm.at[pl.ds(0, _CHUNK)], rows_b[b],
                              semg[b]).wait()

    def _pair_slots(rows_v, j, t):
        q = 16 * t
        raw = rows_v[j, q // 128, q % 128: q % 128 + 16]
        va = lax.bitcast_convert_type(lax.shift_left(raw, 16), jnp.float32)
        vb = lax.bitcast_convert_type(raw & jnp.int32(-65536), jnp.float32)
        return va, vb

    def _wait_scat(b):
        pltpu.make_async_copy(msg_b[b], agg_sh.at[didx_b[b]],
                              sems[b]).wait()

    def _compute(i, b):
        rows_v = rows_b[b]
        ea_v = ea_b[b]
        msg_v = msg_b[b]
        didx_v = didx_b[b]

        def _group(g, gcarry):
            dgrp = dst_all[pl.ds(i * _CHUNK + 16 * g, 16)]
            didx_v[pl.ds(16 * g, 16)] = lax.shift_right_logical(dgrp, 3)
            slot = (dgrp & 7) * 16
            for e in range(16):
                j = 16 * g + e
                arow = ea_v[2 * g + e // 8, (e % 8) * 16: (e % 8) * 16 + 16]
                va, vb = _pair_slots(rows_v, j, 0)
                m = va + arow[0] * vb
                for t in range(1, 8):
                    va, vb = _pair_slots(rows_v, j, t)
                    m = m + arow[2 * t - 1] * va + arow[2 * t] * vb
                va, _ = _pair_slots(rows_v, j, 8)
                m = m + arow[15] * va
                for t in range(8):
                    msg_v[j, 16 * t: 16 * t + 16] = z16
                msg_v[j, pl.ds(slot[e], 16)] = m
            return gcarry
        lax.fori_loop(0, _GROUPS, _group, 0)
        pltpu.async_copy(msg_v, agg_sh.at[didx_v], sems[b], add=True)

    _start_in(0, 0)

    def _pair(i2, carry):
        a = 2 * i2
        b = a + 1

        @pl.when(b < nch)
        def _():
            _start_in(b, 1)
        _wait_in(0)

        @pl.when(i2 >= 1)
        def _():
            _wait_scat(0)
        _compute(a, 0)

        @pl.when(a + 2 < nch)
        def _():
            _start_in(a + 2, 0)

        @pl.when(b < nch)
        def _():
            _wait_in(1)

            @pl.when(i2 >= 1)
            def _():
                _wait_scat(1)
            _compute(b, 1)
        return carry

    lax.fori_loop(0, (nch + 1) // 2, _pair, 0)
    _wait_scat(0)
    _wait_scat(1)
    plsc.subcore_barrier()

    pltpu.sync_copy(agg_sh.at[pl.ds(s * _AR_TILE, _AR_TILE)], nbuf_v)
    pltpu.sync_copy(nbuf_v, out_hbm.at[c, pl.ds(s * _AR_TILE, _AR_TILE)])


def _edge_pass(table, src, dst, ea2):
    mesh = plsc.VectorSubcoreMesh(core_axis_name="c", subcore_axis_name="s")
    f = pl.kernel(
        _edge_body,
        out_type=jax.ShapeDtypeStruct((2, _AROWS, 128), jnp.float32),
        mesh=mesh,
        scratch_types=[
            pltpu.VMEM((_MAXCH * _CHUNK,), jnp.int32),
            pltpu.VMEM((_MAXCH * _CHUNK,), jnp.int32),
            pltpu.VMEM((_AR_TILE, 128), jnp.float32),
            [pltpu.VMEM((_EA_RPC, 128), jnp.float32) for _ in range(2)],
            [pltpu.VMEM((_CHUNK, _RSL, 128), jnp.int32) for _ in range(2)],
            [pltpu.VMEM((_CHUNK, 128), jnp.float32) for _ in range(2)],
            [pltpu.VMEM((_CHUNK,), jnp.int32) for _ in range(2)],
            [pltpu.SemaphoreType.DMA for _ in range(2)],
            [pltpu.SemaphoreType.DMA for _ in range(2)],
            pltpu.VMEM_SHARED((_AROWS, 128), jnp.float32),
        ],
    )
    out = f(table, src, dst, ea2)
    return out.reshape(2, _NP, 16)


def _build_a(w_e, b_e, cin, cout):
    w = w_e.reshape(_DE, cin, cout).transpose(1, 0, 2)
    w = jnp.pad(w, ((0, 0), (0, 0), (0, 16 - cout)))
    b = jnp.pad(b_e.reshape(cin, cout), ((0, 0), (0, 16 - cout)))
    slots = jnp.concatenate(
        [b[:, None, :], w, jnp.zeros((cin, 1, 16), jnp.float32)], axis=1
    )
    ev = slots[:, 0::2, :].reshape(cin, 144)
    od = slots[:, 1::2, :].reshape(cin, 144)
    half = _ROW // 2
    return jnp.concatenate(
        [jnp.pad(ev, ((0, 0), (0, half - 144))),
         jnp.pad(od, ((0, 0), (0, half - 144)))], axis=1)


def _pad_rows(m, rows):
    return jnp.pad(m, ((0, rows - m.shape[0]), (0, 0)))


def kernel(x, edge_index, edge_attr, batch,
           W_e1, b_e1, root1, bias1,
           W_e2, b_e2, root2, bias2,
           W_e3, b_e3, root3, bias3,
           W_lin, b_lin):
    src = edge_index[0]
    dst = edge_index[1]

    a1 = _build_a(W_e1, b_e1, _D, 8)
    root1p = jnp.pad(root1, ((0, 0), (0, 8)))
    bias1p = jnp.pad(bias1, (0, 8)).reshape(1, 16)

    a2 = _pad_rows(_build_a(W_e2, b_e2, 8, 16), 16)
    root2p = _pad_rows(root2, 16)
    bias2p = bias2.reshape(1, 16)

    a3 = _build_a(W_e3, b_e3, 16, 16)
    bias3p = bias3.reshape(1, 16)

    x_pad = jnp.pad(x, ((0, _NP - _N), (0, 0)))
    batch_pad = jnp.concatenate(
        [batch, jnp.full((_NP - _N,), _G, jnp.int32)])
    batch3d = batch_pad.reshape(_NB, 1, _BN)
    npad = _EPAD - _E
    src = jnp.concatenate([src, jnp.zeros((npad,), jnp.int32)])
    dst = jnp.concatenate([dst, jnp.zeros((npad,), jnp.int32)])
    ea2 = jnp.concatenate(
        [edge_attr, jnp.zeros((npad, _DE), jnp.float32)]
    ).reshape(_EPAD * _DE // 128, 128)
    b_lin2d = b_lin.reshape(1, _C)

    def _as_i32(t):
        return t.reshape(_NP, _RSL, 128)

    t1, r1 = _stage_a(x_pad, a1, root1p, bias1p)
    agg1 = _edge_pass(_as_i32(t1), src, dst, ea2)
    t2, r2 = _stage_b(agg1[0], agg1[1], r1, a2, root2p, bias2p)
    agg2 = _edge_pass(_as_i32(t2), src, dst, ea2)
    t3, r3 = _stage_b(agg2[0], agg2[1], r2, a3, root3, bias3p)
    agg3 = _edge_pass(_as_i32(t3), src, dst, ea2)
    out = _stage_c(agg3[0], agg3[1], r3, batch3d, W_lin, b_lin2d)
    return out


# speedup vs baseline: 1.7702x; 1.1722x over previous
import jax
import jax.numpy as jnp
from jax import lax
from jax.experimental import pallas as pl
from jax.experimental.pallas import tpu as pltpu
from jax.experimental.pallas import tpu_sc as plsc

_N = 10000
_E = 160000
_D = 128
_DE = 16
_G = 64
_C = 10

_ROW = 512
_RSL = 2
_NP = 10240
_NB = 16
_BN = _NP // _NB

_CHUNK = 64
_NCHUNKS = _E // _CHUNK
_NWORKERS = 32
_CHB = _NCHUNKS // _NWORKERS
_CHR = _NCHUNKS % _NWORKERS
_MAXCH = _CHB + 1
_EA_RPC = _CHUNK * _DE // 128
_EPAD = _E + _CHUNK


def _pack_table(t):
    half = _ROW // 2
    lo = lax.bitcast_convert_type(
        t[:, :half].astype(jnp.bfloat16), jnp.uint16).astype(jnp.uint32)
    hi = lax.bitcast_convert_type(
        t[:, half:].astype(jnp.bfloat16), jnp.uint16).astype(jnp.uint32)
    return lax.bitcast_convert_type(lo | (hi << 16), jnp.int32)

def _stage_a_body(x_ref, a_ref, root_ref, bias_ref, t_ref, r_ref):
    x = x_ref[...]
    t_ref[...] = _pack_table(
        jnp.dot(x, a_ref[...], preferred_element_type=jnp.float32))
    r_ref[...] = (
        jnp.dot(x, root_ref[...], preferred_element_type=jnp.float32)
        + bias_ref[...]
    )


def _stage_a(x, a1, root1p, bias1p):
    return pl.pallas_call(
        _stage_a_body,
        grid=(_NB,),
        in_specs=[
            pl.BlockSpec((_BN, _D), lambda i: (i, 0)),
            pl.BlockSpec((_D, _ROW), lambda i: (0, 0)),
            pl.BlockSpec((_D, 16), lambda i: (0, 0)),
            pl.BlockSpec((1, 16), lambda i: (0, 0)),
        ],
        out_specs=[
            pl.BlockSpec((_BN, _ROW // 2), lambda i: (i, 0)),
            pl.BlockSpec((_BN, 16), lambda i: (i, 0)),
        ],
        out_shape=[
            jax.ShapeDtypeStruct((_NP, _ROW // 2), jnp.int32),
            jax.ShapeDtypeStruct((_NP, 16), jnp.float32),
        ],
    )(x, a1, root1p, bias1p)


def _stage_b_body(a0_ref, a1_ref, r_ref, an_ref, rootn_ref, biasn_ref,
                  t_ref, rn_ref):
    h = jax.nn.relu(a0_ref[...] + a1_ref[...] + r_ref[...])
    t_ref[...] = _pack_table(
        jnp.dot(h, an_ref[...], preferred_element_type=jnp.float32))
    rn_ref[...] = (
        jnp.dot(h, rootn_ref[...], preferred_element_type=jnp.float32)
        + biasn_ref[...]
    )


def _stage_b(agg0, agg1, r, a_next, root_next, bias_next):
    return pl.pallas_call(
        _stage_b_body,
        grid=(_NB,),
        in_specs=[
            pl.BlockSpec((_BN, 16), lambda i: (i, 0)),
            pl.BlockSpec((_BN, 16), lambda i: (i, 0)),
            pl.BlockSpec((_BN, 16), lambda i: (i, 0)),
            pl.BlockSpec((16, _ROW), lambda i: (0, 0)),
            pl.BlockSpec((16, 16), lambda i: (0, 0)),
            pl.BlockSpec((1, 16), lambda i: (0, 0)),
        ],
        out_specs=[
            pl.BlockSpec((_BN, _ROW // 2), lambda i: (i, 0)),
            pl.BlockSpec((_BN, 16), lambda i: (i, 0)),
        ],
        out_shape=[
            jax.ShapeDtypeStruct((_NP, _ROW // 2), jnp.int32),
            jax.ShapeDtypeStruct((_NP, 16), jnp.float32),
        ],
    )(agg0, agg1, r, a_next, root_next, bias_next)


def _stage_c_body(a0_ref, a1_ref, r_ref, batch_ref, wl_ref, bl_ref,
                  out_ref, pooled_ref):
    i = pl.program_id(0)

    @pl.when(i == 0)
    def _init():
        pooled_ref[...] = jnp.zeros_like(pooled_ref)

    h = jax.nn.relu(a0_ref[...] + a1_ref[...] + r_ref[...])
    b = batch_ref[0, 0, :]
    seg = lax.broadcasted_iota(jnp.int32, (_G, _BN), 0)
    mask = (seg == b[None, :]).astype(jnp.float32)
    pooled_ref[...] += jnp.dot(mask, h, preferred_element_type=jnp.float32)

    @pl.when(i == _NB - 1)
    def _fin():
        out_ref[...] = (
            jnp.dot(pooled_ref[...], wl_ref[...],
                    preferred_element_type=jnp.float32)
            + bl_ref[...]
        )


def _stage_c(agg0, agg1, r, batch3d, w_lin, b_lin2d):
    return pl.pallas_call(
        _stage_c_body,
        grid=(_NB,),
        in_specs=[
            pl.BlockSpec((_BN, 16), lambda i: (i, 0)),
            pl.BlockSpec((_BN, 16), lambda i: (i, 0)),
            pl.BlockSpec((_BN, 16), lambda i: (i, 0)),
            pl.BlockSpec((1, 1, _BN), lambda i: (i, 0, 0)),
            pl.BlockSpec((16, _C), lambda i: (0, 0)),
            pl.BlockSpec((1, _C), lambda i: (0, 0)),
        ],
        out_specs=pl.BlockSpec((_G, _C), lambda i: (0, 0)),
        out_shape=jax.ShapeDtypeStruct((_G, _C), jnp.float32),
        scratch_shapes=[pltpu.VMEM((_G, 16), jnp.float32)],
    )(agg0, agg1, r, batch3d, w_lin, b_lin2d)


_AROWS = _NP // 8
_AR_TILE = _AROWS // 16
_GROUPS = _CHUNK // 16


def _edge_body(table_hbm, src_hbm, dst_hbm, ea_hbm, out0_hbm, out1_hbm,
               src_all, dst_all, nbuf_v,
               ea_b, rows_b, msg_b, didx_b, semg, sems,
               agg_sh):
    c = lax.axis_index("c")
    s = lax.axis_index("s")
    wid = c * 16 + s
    z16 = jnp.zeros((16,), jnp.float32)

    nch = _CHB + jnp.where(wid < _CHR, 1, 0)
    cstart = _CHB * wid + jnp.minimum(wid, _CHR)
    ebase = pl.multiple_of(cstart * _CHUNK, _CHUNK)

    def _zrow(j, carry):
        for t in range(8):
            nbuf_v[j, 16 * t: 16 * t + 16] = z16
        return carry
    lax.fori_loop(0, _AR_TILE, _zrow, 0)
    pltpu.sync_copy(nbuf_v, agg_sh.at[pl.ds(s * _AR_TILE, _AR_TILE)])
    pltpu.sync_copy(src_hbm.at[pl.ds(ebase, _MAXCH * _CHUNK)], src_all)
    pltpu.sync_copy(dst_hbm.at[pl.ds(ebase, _MAXCH * _CHUNK)], dst_all)
    plsc.subcore_barrier()

    def _start_in(i, b):
        ea_off = pl.multiple_of((cstart + i) * _CHUNK, _CHUNK)
        pltpu.async_copy(ea_hbm.at[pl.ds(ea_off, _CHUNK), :], ea_b[b],
                         semg[b])
        pltpu.async_copy(
            table_hbm.at[src_all.at[pl.ds(i * _CHUNK, _CHUNK)]],
            rows_b[b], semg[b])

    def _wait_in(b):
        pltpu.make_async_copy(ea_hbm.at[pl.ds(0, _CHUNK), :], ea_b[b],
                              semg[b]).wait()
        pltpu.make_async_copy(table_hbm.at[pl.ds(0, _CHUNK)], rows_b[b],
                              semg[b]).wait()

    def _pair_slots(rows_v, j, t):
        q = 16 * t
        raw = rows_v[j, q: q + 16]
        va = lax.bitcast_convert_type(lax.shift_left(raw, 16), jnp.float32)
        vb = lax.bitcast_convert_type(raw & jnp.int32(-65536), jnp.float32)
        return va, vb

    def _wait_scat(b):
        pltpu.make_async_copy(msg_b[b], agg_sh.at[didx_b[b]],
                              sems[b]).wait()

    def _compute(i, b):
        rows_v = rows_b[b]
        ea_v = ea_b[b]
        msg_v = msg_b[b]
        didx_v = didx_b[b]

        def _group(g, gcarry):
            dgrp = dst_all[pl.ds(i * _CHUNK + 16 * g, 16)]
            didx_v[pl.ds(16 * g, 16)] = lax.shift_right_logical(dgrp, 3)
            slot = (dgrp & 7) * 16
            for e in range(16):
                j = 16 * g + e
                arow = ea_v[16 * g + e, 0:16]
                va, vb = _pair_slots(rows_v, j, 0)
                m = va + arow[0] * vb
                for t in range(1, 8):
                    va, vb = _pair_slots(rows_v, j, t)
                    m = m + arow[2 * t - 1] * va + arow[2 * t] * vb
                va, _ = _pair_slots(rows_v, j, 8)
                m = m + arow[15] * va
                for t in range(8):
                    msg_v[j, 16 * t: 16 * t + 16] = z16
                msg_v[j, pl.ds(slot[e], 16)] = m
            return gcarry
        lax.fori_loop(0, _GROUPS, _group, 0)
        pltpu.async_copy(msg_v, agg_sh.at[didx_v], sems[b], add=True)

    _start_in(0, 0)

    def _pair(i2, carry):
        a = 2 * i2
        b = a + 1

        @pl.when(b < nch)
        def _():
            _start_in(b, 1)
        _wait_in(0)

        @pl.when(i2 >= 1)
        def _():
            _wait_scat(0)
        _compute(a, 0)

        @pl.when(a + 2 < nch)
        def _():
            _start_in(a + 2, 0)

        @pl.when(b < nch)
        def _():
            _wait_in(1)

            @pl.when(i2 >= 1)
            def _():
                _wait_scat(1)
            _compute(b, 1)
        return carry

    lax.fori_loop(0, (nch + 1) // 2, _pair, 0)
    _wait_scat(0)
    _wait_scat(1)
    plsc.subcore_barrier()

    pltpu.sync_copy(agg_sh.at[pl.ds(s * _AR_TILE, _AR_TILE)], nbuf_v)

    @pl.when(c == 0)
    def _():
        pltpu.sync_copy(nbuf_v, out0_hbm.at[pl.ds(s * _AR_TILE, _AR_TILE)])

    @pl.when(c == 1)
    def _():
        pltpu.sync_copy(nbuf_v, out1_hbm.at[pl.ds(s * _AR_TILE, _AR_TILE)])


def _edge_pass(table, src, dst, ea2):
    mesh = plsc.VectorSubcoreMesh(core_axis_name="c", subcore_axis_name="s")
    f = pl.kernel(
        _edge_body,
        out_type=[jax.ShapeDtypeStruct((_AROWS, 128), jnp.float32),
                  jax.ShapeDtypeStruct((_AROWS, 128), jnp.float32)],
        mesh=mesh,
        scratch_types=[
            pltpu.VMEM((_MAXCH * _CHUNK,), jnp.int32),
            pltpu.VMEM((_MAXCH * _CHUNK,), jnp.int32),
            pltpu.VMEM((_AR_TILE, 128), jnp.float32),
            [pltpu.VMEM((_CHUNK, _DE), jnp.float32) for _ in range(2)],
            [pltpu.VMEM((_CHUNK, _ROW // 2), jnp.int32) for _ in range(2)],
            [pltpu.VMEM((_CHUNK, 128), jnp.float32) for _ in range(2)],
            [pltpu.VMEM((_CHUNK,), jnp.int32) for _ in range(2)],
            [pltpu.SemaphoreType.DMA for _ in range(2)],
            [pltpu.SemaphoreType.DMA for _ in range(2)],
            pltpu.VMEM_SHARED((_AROWS, 128), jnp.float32),
        ],
    )
    a0, a1 = f(table, src, dst, ea2)
    return a0.reshape(_NP, 16), a1.reshape(_NP, 16)


def _build_a(w_e, b_e, cin, cout):
    w = w_e.reshape(_DE, cin, cout).transpose(1, 0, 2)
    w = jnp.pad(w, ((0, 0), (0, 0), (0, 16 - cout)))
    b = jnp.pad(b_e.reshape(cin, cout), ((0, 0), (0, 16 - cout)))
    slots = jnp.concatenate(
        [b[:, None, :], w, jnp.zeros((cin, 1, 16), jnp.float32)], axis=1
    )
    ev = slots[:, 0::2, :].reshape(cin, 144)
    od = slots[:, 1::2, :].reshape(cin, 144)
    half = _ROW // 2
    return jnp.concatenate(
        [jnp.pad(ev, ((0, 0), (0, half - 144))),
         jnp.pad(od, ((0, 0), (0, half - 144)))], axis=1)


def _pad_rows(m, rows):
    return jnp.pad(m, ((0, rows - m.shape[0]), (0, 0)))


def kernel(x, edge_index, edge_attr, batch,
           W_e1, b_e1, root1, bias1,
           W_e2, b_e2, root2, bias2,
           W_e3, b_e3, root3, bias3,
           W_lin, b_lin):
    src = edge_index[0]
    dst = edge_index[1]

    a1 = _build_a(W_e1, b_e1, _D, 8)
    root1p = jnp.pad(root1, ((0, 0), (0, 8)))
    bias1p = jnp.pad(bias1, (0, 8)).reshape(1, 16)

    a2 = _pad_rows(_build_a(W_e2, b_e2, 8, 16), 16)
    root2p = _pad_rows(root2, 16)
    bias2p = bias2.reshape(1, 16)

    a3 = _build_a(W_e3, b_e3, 16, 16)
    bias3p = bias3.reshape(1, 16)

    x_pad = jnp.pad(x, ((0, _NP - _N), (0, 0)))
    batch_pad = jnp.concatenate(
        [batch, jnp.full((_NP - _N,), _G, jnp.int32)])
    batch3d = batch_pad.reshape(_NB, 1, _BN)
    npad = _EPAD - _E
    src = jnp.concatenate([src, jnp.zeros((npad,), jnp.int32)])
    dst = jnp.concatenate([dst, jnp.zeros((npad,), jnp.int32)])
    ea2 = edge_attr
    b_lin2d = b_lin.reshape(1, _C)

    t1, r1 = _stage_a(x_pad, a1, root1p, bias1p)
    agg1 = _edge_pass(t1, src, dst, ea2)
    t2, r2 = _stage_b(agg1[0], agg1[1], r1, a2, root2p, bias2p)
    agg2 = _edge_pass(t2, src, dst, ea2)
    t3, r3 = _stage_b(agg2[0], agg2[1], r2, a3, root3, bias3p)
    agg3 = _edge_pass(t3, src, dst, ea2)
    out = _stage_c(agg3[0], agg3[1], r3, batch3d, W_lin, b_lin2d)
    return out


# speedup vs baseline: 1.8223x; 1.0294x over previous
import jax
import jax.numpy as jnp
from jax import lax
from jax.experimental import pallas as pl
from jax.experimental.pallas import tpu as pltpu
from jax.experimental.pallas import tpu_sc as plsc

_N = 10000
_E = 160000
_D = 128
_DE = 16
_G = 64
_C = 10

_ROW = 512
_RSL = 2
_NP = 10240
_NB = 16
_BN = _NP // _NB

_CHUNK = 64
_NCHUNKS = _E // _CHUNK
_NWORKERS = 32
_CHB = _NCHUNKS // _NWORKERS
_CHR = _NCHUNKS % _NWORKERS
_MAXCH = _CHB + 1
_EA_RPC = _CHUNK * _DE // 128
_EPAD = _E + _CHUNK


def _pack_table(t):
    half = t.shape[1] // 2
    lo = lax.bitcast_convert_type(
        t[:, :half].astype(jnp.bfloat16), jnp.uint16).astype(jnp.uint32)
    hi = lax.bitcast_convert_type(
        t[:, half:].astype(jnp.bfloat16), jnp.uint16).astype(jnp.uint32)
    return lax.bitcast_convert_type(lo | (hi << 16), jnp.int32)

def _stage_a_body(x_ref, a_ref, root_ref, bias_ref, t_ref, r_ref):
    x = x_ref[...]
    t_ref[...] = _pack_table(
        jnp.dot(x, a_ref[...], preferred_element_type=jnp.float32))
    r_ref[...] = (
        jnp.dot(x, root_ref[...], preferred_element_type=jnp.float32)
        + bias_ref[...]
    )


def _stage_a(x, a1, root1p, bias1p):
    aw = a1.shape[1]
    return pl.pallas_call(
        _stage_a_body,
        grid=(_NB,),
        in_specs=[
            pl.BlockSpec((_BN, _D), lambda i: (i, 0)),
            pl.BlockSpec((_D, aw), lambda i: (0, 0)),
            pl.BlockSpec((_D, 16), lambda i: (0, 0)),
            pl.BlockSpec((1, 16), lambda i: (0, 0)),
        ],
        out_specs=[
            pl.BlockSpec((_BN, aw // 2), lambda i: (i, 0)),
            pl.BlockSpec((_BN, 16), lambda i: (i, 0)),
        ],
        out_shape=[
            jax.ShapeDtypeStruct((_NP, aw // 2), jnp.int32),
            jax.ShapeDtypeStruct((_NP, 16), jnp.float32),
        ],
    )(x, a1, root1p, bias1p)


def _stage_b_body(a0_ref, a1_ref, r_ref, an_ref, rootn_ref, biasn_ref,
                  t_ref, rn_ref):
    h = jax.nn.relu(a0_ref[...] + a1_ref[...] + r_ref[...])
    t_ref[...] = _pack_table(
        jnp.dot(h, an_ref[...], preferred_element_type=jnp.float32))
    rn_ref[...] = (
        jnp.dot(h, rootn_ref[...], preferred_element_type=jnp.float32)
        + biasn_ref[...]
    )


def _stage_b(agg0, agg1, r, a_next, root_next, bias_next):
    return pl.pallas_call(
        _stage_b_body,
        grid=(_NB,),
        in_specs=[
            pl.BlockSpec((_BN, 16), lambda i: (i, 0)),
            pl.BlockSpec((_BN, 16), lambda i: (i, 0)),
            pl.BlockSpec((_BN, 16), lambda i: (i, 0)),
            pl.BlockSpec((16, _ROW), lambda i: (0, 0)),
            pl.BlockSpec((16, 16), lambda i: (0, 0)),
            pl.BlockSpec((1, 16), lambda i: (0, 0)),
        ],
        out_specs=[
            pl.BlockSpec((_BN, _ROW // 2), lambda i: (i, 0)),
            pl.BlockSpec((_BN, 16), lambda i: (i, 0)),
        ],
        out_shape=[
            jax.ShapeDtypeStruct((_NP, _ROW // 2), jnp.int32),
            jax.ShapeDtypeStruct((_NP, 16), jnp.float32),
        ],
    )(agg0, agg1, r, a_next, root_next, bias_next)


def _stage_c_body(a0_ref, a1_ref, r_ref, batch_ref, wl_ref, bl_ref,
                  out_ref, pooled_ref):
    i = pl.program_id(0)

    @pl.when(i == 0)
    def _init():
        pooled_ref[...] = jnp.zeros_like(pooled_ref)

    h = jax.nn.relu(a0_ref[...] + a1_ref[...] + r_ref[...])
    b = batch_ref[0, 0, :]
    seg = lax.broadcasted_iota(jnp.int32, (_G, _BN), 0)
    mask = (seg == b[None, :]).astype(jnp.float32)
    pooled_ref[...] += jnp.dot(mask, h, preferred_element_type=jnp.float32)

    @pl.when(i == _NB - 1)
    def _fin():
        out_ref[...] = (
            jnp.dot(pooled_ref[...], wl_ref[...],
                    preferred_element_type=jnp.float32)
            + bl_ref[...]
        )


def _stage_c(agg0, agg1, r, batch3d, w_lin, b_lin2d):
    return pl.pallas_call(
        _stage_c_body,
        grid=(_NB,),
        in_specs=[
            pl.BlockSpec((_BN, 16), lambda i: (i, 0)),
            pl.BlockSpec((_BN, 16), lambda i: (i, 0)),
            pl.BlockSpec((_BN, 16), lambda i: (i, 0)),
            pl.BlockSpec((1, 1, _BN), lambda i: (i, 0, 0)),
            pl.BlockSpec((16, _C), lambda i: (0, 0)),
            pl.BlockSpec((1, _C), lambda i: (0, 0)),
        ],
        out_specs=pl.BlockSpec((_G, _C), lambda i: (0, 0)),
        out_shape=jax.ShapeDtypeStruct((_G, _C), jnp.float32),
        scratch_shapes=[pltpu.VMEM((_G, 16), jnp.float32)],
    )(agg0, agg1, r, batch3d, w_lin, b_lin2d)


_AROWS = _NP // 8
_AR_TILE = _AROWS // 16
_GROUPS = _CHUNK // 16


def _edge_body(table_hbm, src_hbm, dst_hbm, ea_hbm, out0_hbm, out1_hbm,
               src_all, dst_all, nbuf_v,
               ea_b, rows_b, msg_b, didx_b, semg, sems,
               agg_sh, l1=False):
    c = lax.axis_index("c")
    s = lax.axis_index("s")
    wid = c * 16 + s
    z16 = jnp.zeros((16,), jnp.float32)

    nch = _CHB + jnp.where(wid < _CHR, 1, 0)
    cstart = _CHB * wid + jnp.minimum(wid, _CHR)
    ebase = pl.multiple_of(cstart * _CHUNK, _CHUNK)

    def _zrow(j, carry):
        for t in range(8):
            nbuf_v[j, 16 * t: 16 * t + 16] = z16
        return carry
    lax.fori_loop(0, _AR_TILE, _zrow, 0)
    pltpu.sync_copy(nbuf_v, agg_sh.at[pl.ds(s * _AR_TILE, _AR_TILE)])
    pltpu.sync_copy(src_hbm.at[pl.ds(ebase, _MAXCH * _CHUNK)], src_all)
    pltpu.sync_copy(dst_hbm.at[pl.ds(ebase, _MAXCH * _CHUNK)], dst_all)
    plsc.subcore_barrier()

    def _start_in(i, b):
        ea_off = pl.multiple_of((cstart + i) * _CHUNK, _CHUNK)
        pltpu.async_copy(ea_hbm.at[pl.ds(ea_off, _CHUNK), :], ea_b[b],
                         semg[b])
        pltpu.async_copy(
            table_hbm.at[src_all.at[pl.ds(i * _CHUNK, _CHUNK)]],
            rows_b[b], semg[b])

    def _wait_in(b):
        pltpu.make_async_copy(ea_hbm.at[pl.ds(0, _CHUNK), :], ea_b[b],
                              semg[b]).wait()
        pltpu.make_async_copy(table_hbm.at[pl.ds(0, _CHUNK)], rows_b[b],
                              semg[b]).wait()

    def _pair_slots(rows_v, j, t):
        q = 16 * t
        raw = rows_v[j, q: q + 16]
        va = lax.bitcast_convert_type(lax.shift_left(raw, 16), jnp.float32)
        vb = lax.bitcast_convert_type(raw & jnp.int32(-65536), jnp.float32)
        return va, vb

    def _wait_scat(b):
        pltpu.make_async_copy(msg_b[b], agg_sh.at[didx_b[b]],
                              sems[b]).wait()

    def _compute(i, b):
        rows_v = rows_b[b]
        ea_v = ea_b[b]
        msg_v = msg_b[b]
        didx_v = didx_b[b]

        iot = lax.iota(jnp.int32, 16)
        m_lo = jnp.where(iot < 8, 1.0, 0.0).astype(jnp.float32)
        m_hi = jnp.where(iot < 8, 0.0, 1.0).astype(jnp.float32)

        def _group(g, gcarry):
            dgrp = dst_all[pl.ds(i * _CHUNK + 16 * g, 16)]
            didx_v[pl.ds(16 * g, 16)] = lax.shift_right_logical(dgrp, 3)
            slot = (dgrp & 7) * 16
            for e in range(16):
                j = 16 * g + e
                arow = ea_v[16 * g + e, 0:16]
                if l1:
                    acc = None
                    for t in range(5):
                        va, vb = _pair_slots(rows_v, j, t)
                        if t == 0:
                            wa = m_lo + arow[1] * m_hi
                        elif t < 4:
                            wa = arow[4 * t - 1] * m_lo + arow[4 * t + 1] * m_hi
                        else:
                            wa = arow[15] * m_lo
                        acc = va * wa if acc is None else acc + va * wa
                        if t < 4:
                            wb = arow[4 * t] * m_lo + arow[4 * t + 2] * m_hi
                            acc = acc + vb * wb
                    m = acc + lax.rev(acc, (0,))
                else:
                    va, vb = _pair_slots(rows_v, j, 0)
                    m = va + arow[0] * vb
                    for t in range(1, 8):
                        va, vb = _pair_slots(rows_v, j, t)
                        m = m + arow[2 * t - 1] * va + arow[2 * t] * vb
                    va, _ = _pair_slots(rows_v, j, 8)
                    m = m + arow[15] * va
                for t in range(8):
                    msg_v[j, 16 * t: 16 * t + 16] = z16
                msg_v[j, pl.ds(slot[e], 16)] = m
            return gcarry
        lax.fori_loop(0, _GROUPS, _group, 0)
        pltpu.async_copy(msg_v, agg_sh.at[didx_v], sems[b], add=True)

    _start_in(0, 0)

    def _pair(i2, carry):
        a = 2 * i2
        b = a + 1

        @pl.when(b < nch)
        def _():
            _start_in(b, 1)
        _wait_in(0)

        @pl.when(i2 >= 1)
        def _():
            _wait_scat(0)
        _compute(a, 0)

        @pl.when(a + 2 < nch)
        def _():
            _start_in(a + 2, 0)

        @pl.when(b < nch)
        def _():
            _wait_in(1)

            @pl.when(i2 >= 1)
            def _():
                _wait_scat(1)
            _compute(b, 1)
        return carry

    lax.fori_loop(0, (nch + 1) // 2, _pair, 0)
    _wait_scat(0)
    _wait_scat(1)
    plsc.subcore_barrier()

    pltpu.sync_copy(agg_sh.at[pl.ds(s * _AR_TILE, _AR_TILE)], nbuf_v)

    @pl.when(c == 0)
    def _():
        pltpu.sync_copy(nbuf_v, out0_hbm.at[pl.ds(s * _AR_TILE, _AR_TILE)])

    @pl.when(c == 1)
    def _():
        pltpu.sync_copy(nbuf_v, out1_hbm.at[pl.ds(s * _AR_TILE, _AR_TILE)])


def _edge_pass(table, src, dst, ea2, l1=False):
    mesh = plsc.VectorSubcoreMesh(core_axis_name="c", subcore_axis_name="s")
    rw = table.shape[1]

    def body(*refs):
        _edge_body(*refs, l1=l1)

    f = pl.kernel(
        body,
        out_type=[jax.ShapeDtypeStruct((_AROWS, 128), jnp.float32),
                  jax.ShapeDtypeStruct((_AROWS, 128), jnp.float32)],
        mesh=mesh,
        scratch_types=[
            pltpu.VMEM((_MAXCH * _CHUNK,), jnp.int32),
            pltpu.VMEM((_MAXCH * _CHUNK,), jnp.int32),
            pltpu.VMEM((_AR_TILE, 128), jnp.float32),
            [pltpu.VMEM((_CHUNK, _DE), jnp.float32) for _ in range(2)],
            [pltpu.VMEM((_CHUNK, rw), jnp.int32) for _ in range(2)],
            [pltpu.VMEM((_CHUNK, 128), jnp.float32) for _ in range(2)],
            [pltpu.VMEM((_CHUNK,), jnp.int32) for _ in range(2)],
            [pltpu.SemaphoreType.DMA for _ in range(2)],
            [pltpu.SemaphoreType.DMA for _ in range(2)],
            pltpu.VMEM_SHARED((_AROWS, 128), jnp.float32),
        ],
    )
    a0, a1 = f(table, src, dst, ea2)
    return a0.reshape(_NP, 16), a1.reshape(_NP, 16)


def _build_a(w_e, b_e, cin, cout):
    w = w_e.reshape(_DE, cin, cout).transpose(1, 0, 2)
    w = jnp.pad(w, ((0, 0), (0, 0), (0, 16 - cout)))
    b = jnp.pad(b_e.reshape(cin, cout), ((0, 0), (0, 16 - cout)))
    slots = jnp.concatenate(
        [b[:, None, :], w, jnp.zeros((cin, 1, 16), jnp.float32)], axis=1
    )
    ev = slots[:, 0::2, :].reshape(cin, 144)
    od = slots[:, 1::2, :].reshape(cin, 144)
    half = _ROW // 2
    return jnp.concatenate(
        [jnp.pad(ev, ((0, 0), (0, half - 144))),
         jnp.pad(od, ((0, 0), (0, half - 144)))], axis=1)


def _build_a1h(w_e, b_e):
    w = w_e.reshape(_DE, _D, 8).transpose(1, 0, 2)
    b = b_e.reshape(_D, 8)
    slots = jnp.concatenate(
        [b[:, None, :], w, jnp.zeros((_D, 1, 8), jnp.float32)], axis=1)
    ev = slots[:, 0::2, :]
    od = slots[:, 1::2, :]

    def flipodd(st):
        sel = (jnp.arange(st.shape[1]) % 2 == 1)[None, :, None]
        return jnp.where(sel, st[:, :, ::-1], st)

    ev = flipodd(ev).reshape(_D, 72)
    od = flipodd(od).reshape(_D, 72)
    return jnp.concatenate(
        [jnp.pad(ev, ((0, 0), (0, 128 - 72))),
         jnp.pad(od, ((0, 0), (0, 128 - 72)))], axis=1)


def _pad_rows(m, rows):
    return jnp.pad(m, ((0, rows - m.shape[0]), (0, 0)))


def kernel(x, edge_index, edge_attr, batch,
           W_e1, b_e1, root1, bias1,
           W_e2, b_e2, root2, bias2,
           W_e3, b_e3, root3, bias3,
           W_lin, b_lin):
    src = edge_index[0]
    dst = edge_index[1]

    a1 = _build_a1h(W_e1, b_e1)
    root1p = jnp.pad(root1, ((0, 0), (0, 8)))
    bias1p = jnp.pad(bias1, (0, 8)).reshape(1, 16)

    a2 = _pad_rows(_build_a(W_e2, b_e2, 8, 16), 16)
    root2p = _pad_rows(root2, 16)
    bias2p = bias2.reshape(1, 16)

    a3 = _build_a(W_e3, b_e3, 16, 16)
    bias3p = bias3.reshape(1, 16)

    x_pad = jnp.pad(x, ((0, _NP - _N), (0, 0)))
    batch_pad = jnp.concatenate(
        [batch, jnp.full((_NP - _N,), _G, jnp.int32)])
    batch3d = batch_pad.reshape(_NB, 1, _BN)
    npad = _EPAD - _E
    src = jnp.concatenate([src, jnp.zeros((npad,), jnp.int32)])
    dst = jnp.concatenate([dst, jnp.zeros((npad,), jnp.int32)])
    ea2 = edge_attr
    b_lin2d = b_lin.reshape(1, _C)

    t1, r1 = _stage_a(x_pad, a1, root1p, bias1p)
    agg1 = _edge_pass(t1, src, dst, ea2, l1=True)
    t2, r2 = _stage_b(agg1[0], agg1[1], r1, a2, root2p, bias2p)
    agg2 = _edge_pass(t2, src, dst, ea2)
    t3, r3 = _stage_b(agg2[0], agg2[1], r2, a3, root3, bias3p)
    agg3 = _edge_pass(t3, src, dst, ea2)
    out = _stage_c(agg3[0], agg3[1], r3, batch3d, W_lin, b_lin2d)
    return out
